# baseline (speedup 1.0000x reference)
import dataclasses
import functools

import jax
import jax.numpy as jnp
from jax import lax
from jax.experimental import pallas as pl
from jax.experimental.pallas import tpu as pltpu
from jax.experimental.pallas import tpu_sc as plsc

N = 10000
NPAD = 10240
NH = NPAD // 2
NSPLIT = 5000
SHIFT = NH - NSPLIT
JLOC = 5056
JGLOB = NH + JLOC
E = 320000
D = 128

NC = 2
NS = 16
NW = NC * NS
CHUNK = 128
BLKCH = 16
BLKE = BLKCH * CHUNK
EPT = 10240
EPAD = NW * EPT
DNCHUNK = EPT // CHUNK
NPS = NPAD // NS
HPS = NH // NS

BM = 1024

_mesh = plsc.VectorSubcoreMesh(core_axis_name="c", subcore_axis_name="s")

_cp = pltpu.CompilerParams()
if "needs_layout_passes" in pltpu.CompilerParams.__dataclass_fields__:
    _cp = dataclasses.replace(_cp, needs_layout_passes=False)


def _sc_degree(dst3):
    @functools.partial(
        pl.kernel,
        out_type=jax.ShapeDtypeStruct((NC, NPAD, D), jnp.float32),
        mesh=_mesh,
        scratch_types=[
            pltpu.VMEM((DNCHUNK, CHUNK), jnp.int32),
            pltpu.VMEM((CHUNK, D), jnp.float32),
            pltpu.VMEM_SHARED((NPAD, D), jnp.float32),
        ],
    )
    def deg_kernel(dst_hbm, out_hbm, idx_v, ones_v, acc_sh):
        c = lax.axis_index("c")
        s = lax.axis_index("s")
        wid = c * NS + s
        pltpu.sync_copy(dst_hbm.at[wid], idx_v)

        ones16 = jnp.ones((16,), jnp.float32)
        zero16 = jnp.zeros((16,), jnp.float32)

        @pl.loop(0, CHUNK)
        def _(r):
            @pl.loop(0, D // 16)
            def _(j):
                ones_v[r, pl.ds(j * 16, 16)] = zero16

        @pl.loop(0, NPS // CHUNK)
        def _(t):
            pltpu.sync_copy(ones_v, acc_sh.at[pl.ds(s * NPS + t * CHUNK, CHUNK)])

        @pl.loop(0, CHUNK)
        def _(r):
            @pl.loop(0, D // 16)
            def _(j):
                ones_v[r, pl.ds(j * 16, 16)] = ones16

        plsc.subcore_barrier()

        @pl.loop(0, DNCHUNK)
        def _(i):
            pltpu.sync_copy(ones_v, acc_sh.at[idx_v.at[i]], add=True)

        plsc.subcore_barrier()
        pltpu.sync_copy(acc_sh.at[pl.ds(s * NPS, NPS)],
                        out_hbm.at[c].at[pl.ds(s * NPS, NPS)])

    return deg_kernel(dst3)


def _sc_partition(sr3, dr3):
    @functools.partial(
        pl.kernel,
        out_type=[
            jax.ShapeDtypeStruct((NW, 4, EPT), jnp.int32),
            jax.ShapeDtypeStruct((NW, 4, EPT), jnp.int32),
            jax.ShapeDtypeStruct((NW, 4, 16), jnp.int32),
        ],
        mesh=_mesh,
        compiler_params=_cp,
        scratch_types=[
            pltpu.VMEM((DNCHUNK, CHUNK), jnp.int32),
            pltpu.VMEM((DNCHUNK, CHUNK), jnp.int32),
            pltpu.VMEM((EPT + 16,), jnp.int32),
            pltpu.VMEM((EPT + 16,), jnp.int32),
            pltpu.VMEM((16,), jnp.int32),
        ],
    )
    def part_kernel(sr_hbm, dr_hbm, qs_hbm, qd_hbm, cnt_hbm,
                    sr_v, dr_v, st_s, st_d, cnt_v):
        c = lax.axis_index("c")
        s = lax.axis_index("s")
        wid = c * NS + s
        pltpu.sync_copy(sr_hbm.at[wid], sr_v)
        pltpu.sync_copy(dr_hbm.at[wid], dr_v)

        junk16 = jnp.full((16,), JLOC, jnp.int32)

        for q in range(4):
            sh, dh = q // 2, q % 2
            slo = sh * NH
            dlo = dh * NH

            def scan_body(v, off, _slo=slo, _dlo=dlo):
                r = v // 8
                g = v % 8
                s16 = sr_v[r, pl.ds(g * 16, 16)]
                d16 = dr_v[r, pl.ds(g * 16, 16)]
                m = ((s16 >= _slo) & (s16 < _slo + NH)
                     & (d16 >= _dlo) & (d16 < _dlo + NH))
                plsc.store_compressed(st_s.at[pl.ds(off, 16)], s16 - _slo, mask=m)
                plsc.store_compressed(st_d.at[pl.ds(off, 16)], d16 - _dlo, mask=m)
                pop = plsc.all_reduce_population_count(m)
                return off + jnp.max(pop)

            off = lax.fori_loop(0, EPT // 16, scan_body, jnp.int32(0))

            end = (off + BLKE - 1) // BLKE * BLKE
            nfill = end - off

            def fill_body(k, off2):
                st_s[pl.ds(off2, 16)] = junk16
                st_d[pl.ds(off2, 16)] = junk16
                return off2 + 16

            lax.fori_loop(0, (nfill + 15) // 16, fill_body, off)

            nblk = end // BLKE
            cnt_v[...] = jnp.broadcast_to(nblk, (16,))
            pltpu.sync_copy(cnt_v, cnt_hbm.at[wid].at[q])
            pltpu.sync_copy(st_s.at[pl.ds(0, EPT)], qs_hbm.at[wid].at[q])
            pltpu.sync_copy(st_d.at[pl.ds(0, EPT)], qd_hbm.at[wid].at[q])

    return part_kernel(sr3, dr3)


def _sc_scatter(ht, qs4, qd4, cnts):
    @functools.partial(
        pl.kernel,
        out_type=jax.ShapeDtypeStruct((NPAD, D), jnp.float32),
        mesh=_mesh,
        scratch_types=[
            pltpu.VMEM((BLKCH, CHUNK), jnp.int32),
            pltpu.VMEM((BLKCH, CHUNK), jnp.int32),
            pltpu.VMEM((CHUNK, D), jnp.float32),
            pltpu.VMEM((CHUNK, D), jnp.float32),
            pltpu.VMEM((16,), jnp.int32),
            pltpu.VMEM_SHARED((NH, D), jnp.float32),
            pltpu.VMEM_SHARED((NH, D), jnp.float32),
            pltpu.SemaphoreType.DMA,
            pltpu.SemaphoreType.DMA,
        ],
    )
    def gs_kernel(h_hbm, qs_hbm, qd_hbm, cnt_hbm, out_hbm,
                  sb_v, db_v, rows0, rows1, cnt_v, ht_sp, acc_sh,
                  gsem0, gsem1):
        c = lax.axis_index("c")
        s = lax.axis_index("s")
        zero16 = jnp.zeros((16,), jnp.float32)

        @pl.loop(0, 64)
        def _(r):
            @pl.loop(0, D // 16)
            def _(j):
                rows0[r, pl.ds(j * 16, 16)] = zero16

        @pl.loop(0, HPS // 64)
        def _(t):
            pltpu.sync_copy(rows0.at[pl.ds(0, 64)],
                            acc_sh.at[pl.ds(s * HPS + t * 64, 64)])

        for ph in range(2):
            pltpu.sync_copy(h_hbm.at[pl.ds(ph * NH + s * HPS, HPS)],
                            ht_sp.at[pl.ds(s * HPS, HPS)])
            plsc.subcore_barrier()

            q = ph * 2 + c
            for tt in range(2):
                t = 2 * s + tt
                pltpu.sync_copy(cnt_hbm.at[t].at[q], cnt_v)
                nblk = cnt_v[...][0]

                @pl.loop(0, nblk)
                def _(blk):
                    pltpu.sync_copy(
                        qs_hbm.at[t].at[q].at[pl.ds(blk * BLKCH, BLKCH)], sb_v)
                    pltpu.sync_copy(
                        qd_hbm.at[t].at[q].at[pl.ds(blk * BLKCH, BLKCH)], db_v)
                    pltpu.async_copy(ht_sp.at[sb_v.at[0]], rows0, gsem0)
                    for j in range(BLKCH):
                        cur, csem = (rows0, gsem0) if j % 2 == 0 else (rows1, gsem1)
                        nxt, nsem = (rows1, gsem1) if j % 2 == 0 else (rows0, gsem0)
                        pltpu.make_async_copy(
                            ht_sp.at[sb_v.at[j]], cur, csem).wait()
                        if j < BLKCH - 1:
                            pltpu.async_copy(ht_sp.at[sb_v.at[j + 1]], nxt, nsem)
                        pltpu.sync_copy(cur, acc_sh.at[db_v.at[j]], add=True)

            plsc.subcore_barrier()

        pltpu.sync_copy(acc_sh.at[pl.ds(s * HPS, HPS)],
                        out_hbm.at[pl.ds(c * NH + s * HPS, HPS)])

    return gs_kernel(ht, qs4, qd4, cnts)


def _tc_first(x, W, deg0, deg1):
    def body(x_ref, w_ref, d0_ref, d1_ref, ht_ref, dis_ref):
        deg = 1.0 + d0_ref[:, 0:1] + d1_ref[:, 0:1]
        dis = lax.rsqrt(deg)
        h = jnp.dot(x_ref[...], w_ref[...], preferred_element_type=jnp.float32)
        ht_ref[...] = h * dis
        dis_ref[...] = jnp.broadcast_to(dis, (BM, 16))

    return pl.pallas_call(
        body,
        grid=(NPAD // BM,),
        in_specs=[
            pl.BlockSpec((BM, D), lambda i: (i, 0)),
            pl.BlockSpec((D, D), lambda i: (0, 0)),
            pl.BlockSpec((BM, D), lambda i: (i, 0)),
            pl.BlockSpec((BM, D), lambda i: (i, 0)),
        ],
        out_specs=[
            pl.BlockSpec((BM, D), lambda i: (i, 0)),
            pl.BlockSpec((BM, 16), lambda i: (i, 0)),
        ],
        out_shape=[
            jax.ShapeDtypeStruct((NPAD, D), jnp.float32),
            jax.ShapeDtypeStruct((NPAD, 16), jnp.float32),
        ],
    )(x, W, deg0, deg1)


def _tc_mid(p, hprev, dis, b, W):
    def body(p_ref, hp_ref, dis_ref, b_ref, w_ref, out_ref):
        disc = dis_ref[:, 0:1]
        z = disc * (p_ref[...] + hp_ref[...]) + b_ref[...]
        a = jnp.where(z >= 0, z, 0.2 * z)
        out_ref[...] = disc * jnp.dot(a, w_ref[...],
                                      preferred_element_type=jnp.float32)

    return pl.pallas_call(
        body,
        grid=(NPAD // BM,),
        in_specs=[
            pl.BlockSpec((BM, D), lambda i: (i, 0)),
            pl.BlockSpec((BM, D), lambda i: (i, 0)),
            pl.BlockSpec((BM, 16), lambda i: (i, 0)),
            pl.BlockSpec((1, D), lambda i: (0, 0)),
            pl.BlockSpec((D, D), lambda i: (0, 0)),
        ],
        out_specs=pl.BlockSpec((BM, D), lambda i: (i, 0)),
        out_shape=jax.ShapeDtypeStruct((NPAD, D), jnp.float32),
    )(p, hprev, dis, b, W)


def _tc_last(p, hprev, dis, b):
    def body(p_ref, hp_ref, dis_ref, b_ref, out_ref):
        disc = dis_ref[:, 0:1]
        z = disc * (p_ref[...] + hp_ref[...]) + b_ref[...]
        out_ref[...] = jnp.tanh(z)

    return pl.pallas_call(
        body,
        grid=(NPAD // BM,),
        in_specs=[
            pl.BlockSpec((BM, D), lambda i: (i, 0)),
            pl.BlockSpec((BM, D), lambda i: (i, 0)),
            pl.BlockSpec((BM, 16), lambda i: (i, 0)),
            pl.BlockSpec((1, D), lambda i: (0, 0)),
        ],
        out_specs=pl.BlockSpec((BM, D), lambda i: (i, 0)),
        out_shape=jax.ShapeDtypeStruct((NPAD, D), jnp.float32),
    )(p, hprev, dis, b)


@jax.jit
def kernel(x, edge_index, W1, b1, W2, b2, W3, b3):
    pad = EPAD - E
    sr = edge_index[0]
    dr = edge_index[1]
    sr = jnp.where(sr >= NSPLIT, sr + SHIFT, sr)
    dr = jnp.where(dr >= NSPLIT, dr + SHIFT, dr)
    sr = jnp.concatenate([sr, jnp.full((pad,), JGLOB, jnp.int32)])
    dr = jnp.concatenate([dr, jnp.full((pad,), JGLOB, jnp.int32)])
    sr3 = sr.reshape(NW, DNCHUNK, CHUNK)
    dr3 = dr.reshape(NW, DNCHUNK, CHUNK)

    zrows = jnp.zeros((SHIFT, D), x.dtype)
    xp = jnp.concatenate([x[:NSPLIT], zrows, x[NSPLIT:], zrows])

    qs, qd, cnts = _sc_partition(sr3, dr3)
    qs4 = qs.reshape(NW, 4, EPT // CHUNK, CHUNK)
    qd4 = qd.reshape(NW, 4, EPT // CHUNK, CHUNK)

    degp = _sc_degree(dr3)
    ht1, dis = _tc_first(xp, W1, degp[0], degp[1])

    p = _sc_scatter(ht1, qs4, qd4, cnts)
    ht2 = _tc_mid(p, ht1, dis, b1.reshape(1, D), W2)

    p = _sc_scatter(ht2, qs4, qd4, cnts)
    ht3 = _tc_mid(p, ht2, dis, b2.reshape(1, D), W3)

    p = _sc_scatter(ht3, qs4, qd4, cnts)
    out = _tc_last(p, ht3, dis, b3.reshape(1, D))
    return jnp.concatenate([out[:NSPLIT], out[NH:NH + N - NSPLIT]])


# speedup vs baseline: 1.2569x; 1.2569x over previous
import dataclasses
import functools

import jax
import jax.numpy as jnp
from jax import lax
from jax.experimental import pallas as pl
from jax.experimental.pallas import tpu as pltpu
from jax.experimental.pallas import tpu_sc as plsc

N = 10000
NPAD = 10240
NH = NPAD // 2
NSPLIT = 5000
SHIFT = NH - NSPLIT
JLOC = 5056
JGLOB = NH + JLOC
E = 320000
D = 128

NC = 2
NS = 16
NW = NC * NS
CHUNK = 128
BLKCH = 8
BLKE = BLKCH * CHUNK
EPT = 10240
EPAD = NW * EPT
DNCHUNK = EPT // CHUNK
NPS = NPAD // NS
HPS = NH // NS

BM = 1024

_mesh = plsc.VectorSubcoreMesh(core_axis_name="c", subcore_axis_name="s")

_cp = pltpu.CompilerParams()
if "needs_layout_passes" in pltpu.CompilerParams.__dataclass_fields__:
    _cp = dataclasses.replace(_cp, needs_layout_passes=False)


def _sc_degree(dst3):
    @functools.partial(
        pl.kernel,
        out_type=jax.ShapeDtypeStruct((NC, NPAD, D), jnp.float32),
        mesh=_mesh,
        scratch_types=[
            pltpu.VMEM((DNCHUNK, CHUNK), jnp.int32),
            pltpu.VMEM((CHUNK, D), jnp.float32),
            pltpu.VMEM_SHARED((NPAD, D), jnp.float32),
        ],
    )
    def deg_kernel(dst_hbm, out_hbm, idx_v, ones_v, acc_sh):
        c = lax.axis_index("c")
        s = lax.axis_index("s")
        wid = c * NS + s
        pltpu.sync_copy(dst_hbm.at[wid], idx_v)

        ones16 = jnp.ones((16,), jnp.float32)
        zero16 = jnp.zeros((16,), jnp.float32)

        @pl.loop(0, CHUNK)
        def _(r):
            @pl.loop(0, D // 16)
            def _(j):
                ones_v[r, pl.ds(j * 16, 16)] = zero16

        @pl.loop(0, NPS // CHUNK)
        def _(t):
            pltpu.sync_copy(ones_v, acc_sh.at[pl.ds(s * NPS + t * CHUNK, CHUNK)])

        @pl.loop(0, CHUNK)
        def _(r):
            @pl.loop(0, D // 16)
            def _(j):
                ones_v[r, pl.ds(j * 16, 16)] = ones16

        plsc.subcore_barrier()

        @pl.loop(0, DNCHUNK)
        def _(i):
            pltpu.sync_copy(ones_v, acc_sh.at[idx_v.at[i]], add=True)

        plsc.subcore_barrier()
        pltpu.sync_copy(acc_sh.at[pl.ds(s * NPS, NPS)],
                        out_hbm.at[c].at[pl.ds(s * NPS, NPS)])

    return deg_kernel(dst3)


def _sc_partition(sr3, dr3):
    @functools.partial(
        pl.kernel,
        out_type=[
            jax.ShapeDtypeStruct((NW, 4, EPT), jnp.int32),
            jax.ShapeDtypeStruct((NW, 4, EPT), jnp.int32),
            jax.ShapeDtypeStruct((NW, 4, 16), jnp.int32),
        ],
        mesh=_mesh,
        compiler_params=_cp,
        scratch_types=[
            pltpu.VMEM((DNCHUNK, CHUNK), jnp.int32),
            pltpu.VMEM((DNCHUNK, CHUNK), jnp.int32),
            pltpu.VMEM((EPT + 16,), jnp.int32),
            pltpu.VMEM((EPT + 16,), jnp.int32),
            pltpu.VMEM((16,), jnp.int32),
        ],
    )
    def part_kernel(sr_hbm, dr_hbm, qs_hbm, qd_hbm, cnt_hbm,
                    sr_v, dr_v, st_s, st_d, cnt_v):
        c = lax.axis_index("c")
        s = lax.axis_index("s")
        wid = c * NS + s
        pltpu.sync_copy(sr_hbm.at[wid], sr_v)
        pltpu.sync_copy(dr_hbm.at[wid], dr_v)

        junk16 = jnp.full((16,), JLOC, jnp.int32)

        for q in range(4):
            sh, dh = q // 2, q % 2
            slo = sh * NH
            dlo = dh * NH

            def scan_body(v, off, _slo=slo, _dlo=dlo):
                r = v // 8
                g = v % 8
                s16 = sr_v[r, pl.ds(g * 16, 16)]
                d16 = dr_v[r, pl.ds(g * 16, 16)]
                m = ((s16 >= _slo) & (s16 < _slo + NH)
                     & (d16 >= _dlo) & (d16 < _dlo + NH))
                plsc.store_compressed(st_s.at[pl.ds(off, 16)], s16 - _slo, mask=m)
                plsc.store_compressed(st_d.at[pl.ds(off, 16)], d16 - _dlo, mask=m)
                pop = plsc.all_reduce_population_count(m)
                return off + jnp.max(pop)

            off = lax.fori_loop(0, EPT // 16, scan_body, jnp.int32(0))

            end = (off + BLKE - 1) // BLKE * BLKE
            nfill = end - off

            def fill_body(k, off2):
                st_s[pl.ds(off2, 16)] = junk16
                st_d[pl.ds(off2, 16)] = junk16
                return off2 + 16

            lax.fori_loop(0, (nfill + 15) // 16, fill_body, off)

            nblk = end // BLKE
            cnt_v[...] = jnp.broadcast_to(nblk, (16,))
            pltpu.sync_copy(cnt_v, cnt_hbm.at[wid].at[q])
            pltpu.sync_copy(st_s.at[pl.ds(0, EPT)], qs_hbm.at[wid].at[q])
            pltpu.sync_copy(st_d.at[pl.ds(0, EPT)], qd_hbm.at[wid].at[q])

    return part_kernel(sr3, dr3)


def _sc_scatter(ht, qs4, qd4, cnts):
    @functools.partial(
        pl.kernel,
        out_type=jax.ShapeDtypeStruct((NPAD, D), jnp.float32),
        mesh=_mesh,
        scratch_types=[
            pltpu.VMEM((BLKCH, CHUNK), jnp.int32),
            pltpu.VMEM((BLKCH, CHUNK), jnp.int32),
            pltpu.VMEM((CHUNK, D), jnp.float32),
            pltpu.VMEM((CHUNK, D), jnp.float32),
            pltpu.VMEM((16,), jnp.int32),
            pltpu.VMEM_SHARED((NH, D), jnp.float32),
            pltpu.VMEM_SHARED((NH, D), jnp.float32),
            pltpu.SemaphoreType.DMA,
            pltpu.SemaphoreType.DMA,
            pltpu.SemaphoreType.DMA,
            pltpu.SemaphoreType.DMA,
        ],
    )
    def gs_kernel(h_hbm, qs_hbm, qd_hbm, cnt_hbm, out_hbm,
                  sb_v, db_v, rows0, rows1, cnt_v, ht_sp, acc_sh,
                  gsem0, gsem1, ssem0, ssem1):
        c = lax.axis_index("c")
        s = lax.axis_index("s")
        zero16 = jnp.zeros((16,), jnp.float32)

        @pl.loop(0, 64)
        def _(r):
            @pl.loop(0, D // 16)
            def _(j):
                rows0[r, pl.ds(j * 16, 16)] = zero16

        @pl.loop(0, HPS // 64)
        def _(t):
            pltpu.sync_copy(rows0.at[pl.ds(0, 64)],
                            acc_sh.at[pl.ds(s * HPS + t * 64, 64)])

        for ph in range(2):
            pltpu.sync_copy(h_hbm.at[pl.ds(ph * NH + s * HPS, HPS)],
                            ht_sp.at[pl.ds(s * HPS, HPS)])
            plsc.subcore_barrier()

            q = ph * 2 + c
            for tt in range(2):
                t = 2 * s + tt
                pltpu.sync_copy(cnt_hbm.at[t].at[q], cnt_v)
                nblk = cnt_v[...][0]

                @pl.loop(0, nblk)
                def _(blk):
                    pltpu.sync_copy(
                        qs_hbm.at[t].at[q].at[pl.ds(blk * BLKCH, BLKCH)], sb_v)
                    pltpu.sync_copy(
                        qd_hbm.at[t].at[q].at[pl.ds(blk * BLKCH, BLKCH)], db_v)
                    pltpu.async_copy(ht_sp.at[sb_v.at[0]], rows0, gsem0)
                    for j in range(BLKCH):
                        cur, csem, cssem = ((rows0, gsem0, ssem0) if j % 2 == 0
                                            else (rows1, gsem1, ssem1))
                        nxt, nsem, nssem = ((rows1, gsem1, ssem1) if j % 2 == 0
                                            else (rows0, gsem0, gsem0))
                        pltpu.make_async_copy(
                            ht_sp.at[sb_v.at[j]], cur, csem).wait()
                        if j < BLKCH - 1:
                            if j >= 1:
                                psem = ssem1 if j % 2 == 0 else ssem0
                                pltpu.make_async_copy(
                                    nxt, acc_sh.at[db_v.at[j]], psem).wait()
                            pltpu.async_copy(ht_sp.at[sb_v.at[j + 1]], nxt, nsem)
                        pltpu.async_copy(cur, acc_sh.at[db_v.at[j]], cssem,
                                         add=True)
                    pltpu.make_async_copy(
                        rows0, acc_sh.at[db_v.at[0]], ssem0).wait()
                    pltpu.make_async_copy(
                        rows1, acc_sh.at[db_v.at[0]], ssem1).wait()

            plsc.subcore_barrier()

        pltpu.sync_copy(acc_sh.at[pl.ds(s * HPS, HPS)],
                        out_hbm.at[pl.ds(c * NH + s * HPS, HPS)])

    return gs_kernel(ht, qs4, qd4, cnts)


def _tc_first(x, W, deg0, deg1):
    def body(x_ref, w_ref, d0_ref, d1_ref, ht_ref, dis_ref):
        deg = 1.0 + d0_ref[:, 0:1] + d1_ref[:, 0:1]
        dis = lax.rsqrt(deg)
        h = jnp.dot(x_ref[...], w_ref[...], preferred_element_type=jnp.float32)
        ht_ref[...] = h * dis
        dis_ref[...] = jnp.broadcast_to(dis, (BM, 16))

    return pl.pallas_call(
        body,
        grid=(NPAD // BM,),
        in_specs=[
            pl.BlockSpec((BM, D), lambda i: (i, 0)),
            pl.BlockSpec((D, D), lambda i: (0, 0)),
            pl.BlockSpec((BM, D), lambda i: (i, 0)),
            pl.BlockSpec((BM, D), lambda i: (i, 0)),
        ],
        out_specs=[
            pl.BlockSpec((BM, D), lambda i: (i, 0)),
            pl.BlockSpec((BM, 16), lambda i: (i, 0)),
        ],
        out_shape=[
            jax.ShapeDtypeStruct((NPAD, D), jnp.float32),
            jax.ShapeDtypeStruct((NPAD, 16), jnp.float32),
        ],
    )(x, W, deg0, deg1)


def _tc_mid(p, hprev, dis, b, W):
    def body(p_ref, hp_ref, dis_ref, b_ref, w_ref, out_ref):
        disc = dis_ref[:, 0:1]
        z = disc * (p_ref[...] + hp_ref[...]) + b_ref[...]
        a = jnp.where(z >= 0, z, 0.2 * z)
        out_ref[...] = disc * jnp.dot(a, w_ref[...],
                                      preferred_element_type=jnp.float32)

    return pl.pallas_call(
        body,
        grid=(NPAD // BM,),
        in_specs=[
            pl.BlockSpec((BM, D), lambda i: (i, 0)),
            pl.BlockSpec((BM, D), lambda i: (i, 0)),
            pl.BlockSpec((BM, 16), lambda i: (i, 0)),
            pl.BlockSpec((1, D), lambda i: (0, 0)),
            pl.BlockSpec((D, D), lambda i: (0, 0)),
        ],
        out_specs=pl.BlockSpec((BM, D), lambda i: (i, 0)),
        out_shape=jax.ShapeDtypeStruct((NPAD, D), jnp.float32),
    )(p, hprev, dis, b, W)


def _tc_last(p, hprev, dis, b):
    def body(p_ref, hp_ref, dis_ref, b_ref, out_ref):
        disc = dis_ref[:, 0:1]
        z = disc * (p_ref[...] + hp_ref[...]) + b_ref[...]
        out_ref[...] = jnp.tanh(z)

    return pl.pallas_call(
        body,
        grid=(NPAD // BM,),
        in_specs=[
            pl.BlockSpec((BM, D), lambda i: (i, 0)),
            pl.BlockSpec((BM, D), lambda i: (i, 0)),
            pl.BlockSpec((BM, 16), lambda i: (i, 0)),
            pl.BlockSpec((1, D), lambda i: (0, 0)),
        ],
        out_specs=pl.BlockSpec((BM, D), lambda i: (i, 0)),
        out_shape=jax.ShapeDtypeStruct((NPAD, D), jnp.float32),
    )(p, hprev, dis, b)


@jax.jit
def kernel(x, edge_index, W1, b1, W2, b2, W3, b3):
    pad = EPAD - E
    sr = edge_index[0]
    dr = edge_index[1]
    sr = jnp.where(sr >= NSPLIT, sr + SHIFT, sr)
    dr = jnp.where(dr >= NSPLIT, dr + SHIFT, dr)
    sr = jnp.concatenate([sr, jnp.full((pad,), JGLOB, jnp.int32)])
    dr = jnp.concatenate([dr, jnp.full((pad,), JGLOB, jnp.int32)])
    sr3 = sr.reshape(NW, DNCHUNK, CHUNK)
    dr3 = dr.reshape(NW, DNCHUNK, CHUNK)

    zrows = jnp.zeros((SHIFT, D), x.dtype)
    xp = jnp.concatenate([x[:NSPLIT], zrows, x[NSPLIT:], zrows])

    qs, qd, cnts = _sc_partition(sr3, dr3)
    qs4 = qs.reshape(NW, 4, EPT // CHUNK, CHUNK)
    qd4 = qd.reshape(NW, 4, EPT // CHUNK, CHUNK)

    degp = _sc_degree(dr3)
    ht1, dis = _tc_first(xp, W1, degp[0], degp[1])

    p = _sc_scatter(ht1, qs4, qd4, cnts)
    ht2 = _tc_mid(p, ht1, dis, b1.reshape(1, D), W2)

    p = _sc_scatter(ht2, qs4, qd4, cnts)
    ht3 = _tc_mid(p, ht2, dis, b2.reshape(1, D), W3)

    p = _sc_scatter(ht3, qs4, qd4, cnts)
    out = _tc_last(p, ht3, dis, b3.reshape(1, D))
    return jnp.concatenate([out[:NSPLIT], out[NH:NH + N - NSPLIT]])
